# baseline (speedup 1.0000x reference)
import functools

import jax
import jax.numpy as jnp
from jax import lax
from jax.experimental import pallas as pl
from jax.experimental.pallas import tpu as pltpu
from jax.experimental.pallas import tpu_sc as plsc

NUM_EXPERTS_C = 8
TOP_K_C = 2
BT = 256


def _router_body(x_ref, g_ref, id0_ref, id1_ref, w0_ref, w1_ref):
    x = x_ref[...]
    g = g_ref[...]
    logits = jax.lax.dot_general(
        x, g, (((1,), (1,)), ((), ())), preferred_element_type=jnp.float32)
    rb, e = logits.shape
    iota = jax.lax.broadcasted_iota(jnp.int32, (rb, e), 1)
    m0 = jnp.max(logits, axis=-1, keepdims=True)
    am0 = jnp.min(jnp.where(logits == m0, iota, e), axis=-1, keepdims=True)
    l2 = jnp.where(iota == am0, -jnp.inf, logits)
    m1 = jnp.max(l2, axis=-1, keepdims=True)
    am1 = jnp.min(jnp.where(l2 == m1, iota, e), axis=-1, keepdims=True)
    t = jnp.exp(m1 - m0)
    w0 = 1.0 / (1.0 + t)
    id0_ref[...] = am0
    id1_ref[...] = am1
    w0_ref[...] = w0
    w1_ref[...] = t * w0


def _run_router(hidden_states, gate_w):
    T, H = hidden_states.shape
    E = gate_w.shape[0]
    RB = 1024
    grid = (T // RB,)
    out_shapes = (
        jax.ShapeDtypeStruct((T, 1), jnp.int32),
        jax.ShapeDtypeStruct((T, 1), jnp.int32),
        jax.ShapeDtypeStruct((T, 1), jnp.float32),
        jax.ShapeDtypeStruct((T, 1), jnp.float32),
    )
    o_spec = pl.BlockSpec((RB, 1), lambda i: (i, 0))
    return pl.pallas_call(
        _router_body,
        grid=grid,
        in_specs=[
            pl.BlockSpec((RB, H), lambda i: (i, 0)),
            pl.BlockSpec((E, H), lambda i: (0, 0)),
        ],
        out_specs=(o_spec, o_spec, o_spec, o_spec),
        out_shape=out_shapes,
    )(hidden_states, gate_w)


def _bookkeeping(id0, id1, w0, w1, T, E, NBmax):
    eflat = jnp.concatenate([id0[:, 0], id1[:, 0]])
    wflat = jnp.concatenate([w0[:, 0], w1[:, 0]])
    A = eflat.shape[0]
    oh = (eflat[:, None] == jnp.arange(E, dtype=jnp.int32)[None, :]).astype(jnp.int32)
    counts = jnp.sum(oh, axis=0)
    rank = jnp.take_along_axis(jnp.cumsum(oh, axis=0), eflat[:, None], axis=1)[:, 0] - 1
    padded = ((counts + BT - 1) // BT) * BT
    pstart = jnp.concatenate([jnp.zeros((1,), jnp.int32),
                              jnp.cumsum(padded)[:-1].astype(jnp.int32)])
    pos = pstart[eflat] + rank
    nbe = padded // BT
    bstart = jnp.cumsum(nbe).astype(jnp.int32)
    nb = bstart[-1]
    blk = jnp.minimum(jnp.arange(NBmax, dtype=jnp.int32), nb - 1)
    bexp = jnp.searchsorted(bstart, blk, side='right').astype(jnp.int32)
    S = NBmax * BT
    tok = jnp.concatenate([jnp.arange(T, dtype=jnp.int32)] * 2)
    tok_slot = jnp.zeros((S,), jnp.int32).at[pos].set(tok)
    w_slot = jnp.zeros((S,), jnp.float32).at[pos].set(wflat)
    return bexp, nb, tok_slot, w_slot, pos[:T], pos[T:]


def _mlp_body(bexp_ref, nb_ref, x_ref, w1_ref, w3_ref, w2_ref,
              b13_ref, b2_ref, ws_ref, y_ref):
    I = w1_ref.shape[1]

    @pl.when(pl.program_id(0) < nb_ref[0])
    def _():
        x = x_ref[0]
        a = jax.lax.dot_general(
            x, w1_ref[0], (((1,), (1,)), ((), ())),
            preferred_element_type=jnp.float32) + b13_ref[0, :, :I]
        c = jax.lax.dot_general(
            x, w3_ref[0], (((1,), (1,)), ((), ())),
            preferred_element_type=jnp.float32) + b13_ref[0, :, I:]
        h = a * jax.lax.logistic(a) * c
        acc = jax.lax.dot_general(
            h, w2_ref[0], (((1,), (1,)), ((), ())),
            preferred_element_type=jnp.float32)
        y_ref[0] = (acc + b2_ref[0]) * ws_ref[0]


def _run_mlp(x_sorted, w1, w3, w2, w13_bias, w2_bias, w_slot, bexp, nb, NBmax):
    E, I, H = w1.shape
    S = NBmax * BT
    x3 = x_sorted.reshape(NBmax, BT, H)
    ws3 = w_slot.reshape(NBmax, BT, 1)
    nb_arr = jnp.reshape(nb, (1,)).astype(jnp.int32)

    def live(b, bexp_r, nb_r):
        return jnp.minimum(b, nb_r[0] - 1)

    def xmap(b, bexp_r, nb_r):
        return (live(b, bexp_r, nb_r), 0, 0)

    def wmap(b, bexp_r, nb_r):
        return (bexp_r[live(b, bexp_r, nb_r)], 0, 0)

    def bmap3(b, bexp_r, nb_r):
        return (bexp_r[live(b, bexp_r, nb_r)], 0, 0)

    grid_spec = pltpu.PrefetchScalarGridSpec(
        num_scalar_prefetch=2,
        grid=(NBmax,),
        in_specs=[
            pl.BlockSpec((1, BT, H), xmap),
            pl.BlockSpec((1, I, H), wmap),
            pl.BlockSpec((1, I, H), wmap),
            pl.BlockSpec((1, H, I), wmap),
            pl.BlockSpec((1, 1, 2 * I), bmap3),
            pl.BlockSpec((1, 1, H), bmap3),
            pl.BlockSpec((1, BT, 1), xmap),
        ],
        out_specs=pl.BlockSpec((1, BT, H), xmap),
    )
    y3 = pl.pallas_call(
        _mlp_body,
        grid_spec=grid_spec,
        out_shape=jax.ShapeDtypeStruct((NBmax, BT, H), jnp.float32),
        compiler_params=pltpu.CompilerParams(
            dimension_semantics=("arbitrary",),
            vmem_limit_bytes=120 * 1024 * 1024,
        ),
    )(bexp, nb_arr, x3, w1, w3, w2,
      w13_bias.reshape(E, 1, 2 * I), w2_bias.reshape(E, 1, H), ws3)
    return y3.reshape(S, H)


_NW = 32


def _sc_gather(hs, tok_slot, S):
    T, H = hs.shape
    per_w = S // _NW
    CH = 104
    nch = per_w // CH
    mesh = plsc.VectorSubcoreMesh(core_axis_name="c", subcore_axis_name="s")

    @functools.partial(
        pl.kernel, mesh=mesh,
        out_type=jax.ShapeDtypeStruct((S, H), jnp.float32),
        scratch_types=[
            pltpu.VMEM((CH,), jnp.int32),
            pltpu.VMEM((CH, H), jnp.float32),
            pltpu.SemaphoreType.DMA,
        ],
    )
    def k(hs_hbm, idx_hbm, out_hbm, idx_v, rows_v, sem):
        wid = lax.axis_index("s") * 2 + lax.axis_index("c")
        base = wid * per_w
        for c in range(nch):
            off = base + c * CH
            pltpu.sync_copy(idx_hbm.at[pl.ds(off, CH)], idx_v)
            pltpu.async_copy(hs_hbm.at[idx_v], rows_v, sem).wait()
            pltpu.sync_copy(rows_v, out_hbm.at[pl.ds(off, CH)])

    return k(hs, tok_slot)


def _sc_combine(y, pos0, pos1):
    S, H = y.shape
    T = pos0.shape[0]
    per_w = T // _NW
    CH = 32
    nch = per_w // CH
    nv = H // 16
    mesh = plsc.VectorSubcoreMesh(core_axis_name="c", subcore_axis_name="s")

    @functools.partial(
        pl.kernel, mesh=mesh,
        out_type=jax.ShapeDtypeStruct((T, H), jnp.float32),
        scratch_types=[
            pltpu.VMEM((CH,), jnp.int32),
            pltpu.VMEM((CH,), jnp.int32),
            pltpu.VMEM((CH, H), jnp.float32),
            pltpu.VMEM((CH, H), jnp.float32),
            pltpu.SemaphoreType.DMA,
        ],
    )
    def k(y_hbm, p0_hbm, p1_hbm, out_hbm, i0_v, i1_v, a_v, b_v, sem):
        wid = lax.axis_index("s") * 2 + lax.axis_index("c")
        base = wid * per_w
        for c in range(nch):
            off = base + c * CH
            pltpu.sync_copy(p0_hbm.at[pl.ds(off, CH)], i0_v)
            pltpu.sync_copy(p1_hbm.at[pl.ds(off, CH)], i1_v)
            pltpu.async_copy(y_hbm.at[i0_v], a_v, sem).wait()
            pltpu.async_copy(y_hbm.at[i1_v], b_v, sem).wait()

            def row(r, _):
                def col(j8, _):
                    for jj in range(8):
                        sl = pl.ds((j8 * 8 + jj) * 16, 16)
                        a_v[r, sl] = a_v[r, sl] + b_v[r, sl]
                    return 0
                return lax.fori_loop(0, nv // 8, col, 0)

            lax.fori_loop(0, CH, row, 0)
            pltpu.sync_copy(a_v, out_hbm.at[pl.ds(off, CH)])

    return k(y, pos0, pos1)


def kernel(hidden_states, gate_w, w1, w3, w2, w13_bias, w2_bias):
    T, H = hidden_states.shape
    E = w1.shape[0]
    A = T * TOP_K_C
    NBmax = A // BT + (E - 1)

    id0, id1, w0, w1r, = _run_router(hidden_states, gate_w)
    bexp, nb, tok_slot, w_slot, pos0, pos1 = _bookkeeping(
        id0, id1, w0, w1r, T, E, NBmax)

    x_sorted = _sc_gather(hidden_states, tok_slot, NBmax * BT)
    y = _run_mlp(x_sorted, w1, w3, w2, w13_bias, w2_bias,
                 w_slot, bexp, nb, NBmax)
    return _sc_combine(y, pos0, pos1)


# speedup vs baseline: 1.3312x; 1.3312x over previous
import functools

import jax
import jax.numpy as jnp
from jax import lax
from jax.experimental import pallas as pl
from jax.experimental.pallas import tpu as pltpu
from jax.experimental.pallas import tpu_sc as plsc

NUM_EXPERTS_C = 8
TOP_K_C = 2
BT = 256


def _router_body(x_ref, g_ref, id0_ref, id1_ref, w0_ref, w1_ref):
    x = x_ref[...]
    g = g_ref[...]
    logits = jax.lax.dot_general(
        x, g, (((1,), (1,)), ((), ())), preferred_element_type=jnp.float32)
    rb, e = logits.shape
    iota = jax.lax.broadcasted_iota(jnp.int32, (rb, e), 1)
    m0 = jnp.max(logits, axis=-1, keepdims=True)
    am0 = jnp.min(jnp.where(logits == m0, iota, e), axis=-1, keepdims=True)
    l2 = jnp.where(iota == am0, -jnp.inf, logits)
    m1 = jnp.max(l2, axis=-1, keepdims=True)
    am1 = jnp.min(jnp.where(l2 == m1, iota, e), axis=-1, keepdims=True)
    t = jnp.exp(m1 - m0)
    w0 = 1.0 / (1.0 + t)
    id0_ref[...] = am0
    id1_ref[...] = am1
    w0_ref[...] = w0
    w1_ref[...] = t * w0


def _run_router(hidden_states, gate_w):
    T, H = hidden_states.shape
    E = gate_w.shape[0]
    RB = 1024
    grid = (T // RB,)
    out_shapes = (
        jax.ShapeDtypeStruct((T, 1), jnp.int32),
        jax.ShapeDtypeStruct((T, 1), jnp.int32),
        jax.ShapeDtypeStruct((T, 1), jnp.float32),
        jax.ShapeDtypeStruct((T, 1), jnp.float32),
    )
    o_spec = pl.BlockSpec((RB, 1), lambda i: (i, 0))
    return pl.pallas_call(
        _router_body,
        grid=grid,
        in_specs=[
            pl.BlockSpec((RB, H), lambda i: (i, 0)),
            pl.BlockSpec((E, H), lambda i: (0, 0)),
        ],
        out_specs=(o_spec, o_spec, o_spec, o_spec),
        out_shape=out_shapes,
    )(hidden_states, gate_w)


def _sc_route(eflat, E):
    A = eflat.shape[0]
    NSUB = 16
    C = A // NSUB
    NV = C // 16
    mesh = plsc.VectorSubcoreMesh(core_axis_name="c", subcore_axis_name="s",
                                  num_cores=1)

    @functools.partial(
        pl.kernel, mesh=mesh,
        compiler_params=pltpu.CompilerParams(needs_layout_passes=False),
        out_type=(
            jax.ShapeDtypeStruct((A,), jnp.int32),
            jax.ShapeDtypeStruct((16,), jnp.int32),
            jax.ShapeDtypeStruct((NSUB, 16), jnp.int32),
        ),
        scratch_types=[
            pltpu.VMEM((C,), jnp.int32),
            pltpu.VMEM((C,), jnp.int32),
            pltpu.VMEM((16,), jnp.int32),
            pltpu.VMEM((16,), jnp.int32),
            pltpu.VMEM((NSUB, 16), jnp.int32),
        ],
    )
    def k(e_hbm, pos_hbm, meta_hbm, grid_hbm,
          e_v, pos_v, st_a, st_b, grid_v):
        wid = lax.axis_index("s")
        base = wid * C
        pltpu.sync_copy(e_hbm.at[pl.ds(base, C)], e_v)
        iota = lax.broadcasted_iota(jnp.int32, (16,), 0)
        zero = jnp.zeros((16,), jnp.int32)
        run = [zero for _ in range(E)]
        for v in range(NV):
            ev = e_v[pl.ds(v * 16, 16)]
            rank = zero
            for e in range(E):
                m = ev == e
                pc = jnp.cumsum(jnp.where(m, 1, 0))
                rank = jnp.where(m, run[e] + pc - 1, rank)
                run[e] = run[e] + plsc.all_reduce_population_count(m)
            pos_v[pl.ds(v * 16, 16)] = rank
        cnt16 = zero
        for e in range(E):
            cnt16 = jnp.where(iota == e, run[e], cnt16)
        st_a[...] = cnt16
        for w in range(NSUB):
            @pl.when(wid == w)
            def _(w=w):
                pltpu.sync_copy(st_a, grid_hbm.at[w])
        plsc.subcore_barrier()
        pltpu.sync_copy(grid_hbm, grid_v)
        tot = zero
        prior = zero
        widv = jnp.full((16,), wid, jnp.int32)
        for w in range(NSUB):
            gv = grid_v[w]
            tot = tot + gv
            prior = prior + jnp.where(jnp.full((16,), w, jnp.int32) < widv, gv, zero)
        padded = ((tot + (BT - 1)) // BT) * BT
        csum = jnp.cumsum(padded)
        pstart = csum - padded
        base_v = pstart + prior
        st_a[...] = base_v
        for v in range(NV):
            ev = e_v[pl.ds(v * 16, 16)]
            bse = plsc.load_gather(st_a, [ev])
            pos_v[pl.ds(v * 16, 16)] = pos_v[pl.ds(v * 16, 16)] + bse
        pltpu.sync_copy(pos_v, pos_hbm.at[pl.ds(base, C)])

        @pl.when(wid == 0)
        def _():
            st_b[...] = csum // BT
            pltpu.sync_copy(st_b, meta_hbm)

    pos, meta, _grid = k(eflat)
    return pos, meta


def _sc_gather_scatter(hs, pos, wflat, S):
    T, H = hs.shape
    A = pos.shape[0]
    C = A // _NW
    CH = 64
    nch = C // CH
    mesh = plsc.VectorSubcoreMesh(core_axis_name="c", subcore_axis_name="s")

    @functools.partial(
        pl.kernel, mesh=mesh,
        out_type=(
            jax.ShapeDtypeStruct((S, H), jnp.float32),
            jax.ShapeDtypeStruct((S,), jnp.float32),
        ),
        scratch_types=[
            pltpu.VMEM((CH,), jnp.int32),
            pltpu.VMEM((CH,), jnp.int32),
            pltpu.VMEM((CH,), jnp.float32),
            pltpu.VMEM((CH, H), jnp.float32),
            pltpu.SemaphoreType.DMA,
            pltpu.SemaphoreType.DMA,
        ],
    )
    def k(hs_hbm, pos_hbm, wf_hbm, xs_hbm, ws_hbm,
          idx_v, tok_v, w_v, rows_v, sem_g, sem_s):
        wid = lax.axis_index("s") * 2 + lax.axis_index("c")
        base = wid * C
        iota = lax.broadcasted_iota(jnp.int32, (16,), 0)
        for c in range(nch):
            off = base + c * CH
            pltpu.sync_copy(pos_hbm.at[pl.ds(off, CH)], idx_v)
            pltpu.sync_copy(wf_hbm.at[pl.ds(off, CH)], w_v)
            tb = off - jnp.where(off >= T, T, 0)
            for v in range(CH // 16):
                tok_v[pl.ds(v * 16, 16)] = iota + (tb + v * 16)
            pltpu.async_copy(hs_hbm.at[tok_v], rows_v, sem_g).wait()
            pltpu.async_copy(rows_v, xs_hbm.at[idx_v], sem_s).wait()
            pltpu.async_copy(w_v, ws_hbm.at[idx_v], sem_s).wait()

    return k(hs, pos, wflat)


def _mlp_body(meta_ref, x_ref, w1_ref, w3_ref, w2_ref,
              b13_ref, b2_ref, ws_ref, y_ref):
    I = w1_ref.shape[1]

    @pl.when(pl.program_id(0) < meta_ref[NUM_EXPERTS_C - 1])
    def _():
        x = x_ref[0]
        a = jax.lax.dot_general(
            x, w1_ref[0], (((1,), (1,)), ((), ())),
            preferred_element_type=jnp.float32) + b13_ref[0, :, :I]
        c = jax.lax.dot_general(
            x, w3_ref[0], (((1,), (1,)), ((), ())),
            preferred_element_type=jnp.float32) + b13_ref[0, :, I:]
        h = a * jax.lax.logistic(a) * c
        acc = jax.lax.dot_general(
            h, w2_ref[0], (((1,), (1,)), ((), ())),
            preferred_element_type=jnp.float32)
        y_ref[0] = (acc + b2_ref[0]) * ws_ref[0]


def _run_mlp(x_sorted, w1, w3, w2, w13_bias, w2_bias, w_slot, meta, NBmax):
    E, I, H = w1.shape
    S = NBmax * BT
    x3 = x_sorted.reshape(NBmax, BT, H)
    ws3 = w_slot.reshape(NBmax, BT, 1)

    def live(b, meta_r):
        return jnp.minimum(b, meta_r[E - 1] - 1)

    def expert(b, meta_r):
        bl = live(b, meta_r)
        ex = jnp.int32(0)
        for e in range(E):
            ex = ex + jnp.where(meta_r[e] <= bl, 1, 0).astype(jnp.int32)
        return ex

    def xmap(b, meta_r):
        return (live(b, meta_r), 0, 0)

    def wmap(b, meta_r):
        return (expert(b, meta_r), 0, 0)

    grid_spec = pltpu.PrefetchScalarGridSpec(
        num_scalar_prefetch=1,
        grid=(NBmax,),
        in_specs=[
            pl.BlockSpec((1, BT, H), xmap),
            pl.BlockSpec((1, I, H), wmap),
            pl.BlockSpec((1, I, H), wmap),
            pl.BlockSpec((1, H, I), wmap),
            pl.BlockSpec((1, 1, 2 * I), wmap),
            pl.BlockSpec((1, 1, H), wmap),
            pl.BlockSpec((1, BT, 1), xmap),
        ],
        out_specs=pl.BlockSpec((1, BT, H), xmap),
    )
    y3 = pl.pallas_call(
        _mlp_body,
        grid_spec=grid_spec,
        out_shape=jax.ShapeDtypeStruct((NBmax, BT, H), jnp.float32),
        compiler_params=pltpu.CompilerParams(
            dimension_semantics=("arbitrary",),
            vmem_limit_bytes=120 * 1024 * 1024,
        ),
    )(meta, x3, w1, w3, w2,
      w13_bias.reshape(E, 1, 2 * I), w2_bias.reshape(E, 1, H), ws3)
    return y3.reshape(S, H)


_NW = 32


def _sc_combine(y, pos, T):
    S, H = y.shape
    per_w = T // _NW
    CH = 32
    nch = per_w // CH
    nv = H // 16
    mesh = plsc.VectorSubcoreMesh(core_axis_name="c", subcore_axis_name="s")

    @functools.partial(
        pl.kernel, mesh=mesh,
        out_type=jax.ShapeDtypeStruct((T, H), jnp.float32),
        scratch_types=[
            pltpu.VMEM((CH,), jnp.int32),
            pltpu.VMEM((CH,), jnp.int32),
            pltpu.VMEM((CH, H), jnp.float32),
            pltpu.VMEM((CH, H), jnp.float32),
            pltpu.SemaphoreType.DMA,
        ],
    )
    def k(y_hbm, p_hbm, out_hbm, i0_v, i1_v, a_v, b_v, sem):
        wid = lax.axis_index("s") * 2 + lax.axis_index("c")
        base = wid * per_w
        for c in range(nch):
            off = base + c * CH
            pltpu.sync_copy(p_hbm.at[pl.ds(off, CH)], i0_v)
            pltpu.sync_copy(p_hbm.at[pl.ds(T + off, CH)], i1_v)
            pltpu.async_copy(y_hbm.at[i0_v], a_v, sem).wait()
            pltpu.async_copy(y_hbm.at[i1_v], b_v, sem).wait()

            def row(r, _):
                def col(j8, _):
                    for jj in range(8):
                        sl = pl.ds((j8 * 8 + jj) * 16, 16)
                        a_v[r, sl] = a_v[r, sl] + b_v[r, sl]
                    return 0
                return lax.fori_loop(0, nv // 8, col, 0)

            lax.fori_loop(0, CH, row, 0)
            pltpu.sync_copy(a_v, out_hbm.at[pl.ds(off, CH)])

    return k(y, pos)


def kernel(hidden_states, gate_w, w1, w3, w2, w13_bias, w2_bias):
    T, H = hidden_states.shape
    E = w1.shape[0]
    A = T * TOP_K_C
    NBmax = A // BT + (E - 1)

    id0, id1, w0, w1r = _run_router(hidden_states, gate_w)
    eflat = jnp.concatenate([id0[:, 0], id1[:, 0]])
    wflat = jnp.concatenate([w0[:, 0], w1r[:, 0]])
    pos, meta = _sc_route(eflat, E)
    x_sorted, w_slot = _sc_gather_scatter(hidden_states, pos, wflat, NBmax * BT)
    y = _run_mlp(x_sorted, w1, w3, w2, w13_bias, w2_bias,
                 w_slot, meta, NBmax)
    return _sc_combine(y, pos, T)


# speedup vs baseline: 1.4712x; 1.1052x over previous
import functools

import jax
import jax.numpy as jnp
from jax import lax
from jax.experimental import pallas as pl
from jax.experimental.pallas import tpu as pltpu
from jax.experimental.pallas import tpu_sc as plsc

NUM_EXPERTS_C = 8
TOP_K_C = 2
BT = 256


def _router_body(x_ref, g_ref, id0_ref, id1_ref, w0_ref, w1_ref):
    x = x_ref[...]
    g = g_ref[...]
    logits = jax.lax.dot_general(
        x, g, (((1,), (1,)), ((), ())), preferred_element_type=jnp.float32)
    rb, e = logits.shape
    iota = jax.lax.broadcasted_iota(jnp.int32, (rb, e), 1)
    m0 = jnp.max(logits, axis=-1, keepdims=True)
    am0 = jnp.min(jnp.where(logits == m0, iota, e), axis=-1, keepdims=True)
    l2 = jnp.where(iota == am0, -jnp.inf, logits)
    m1 = jnp.max(l2, axis=-1, keepdims=True)
    am1 = jnp.min(jnp.where(l2 == m1, iota, e), axis=-1, keepdims=True)
    t = jnp.exp(m1 - m0)
    w0 = 1.0 / (1.0 + t)
    id0_ref[...] = am0
    id1_ref[...] = am1
    w0_ref[...] = w0
    w1_ref[...] = t * w0


def _run_router(hidden_states, gate_w):
    T, H = hidden_states.shape
    E = gate_w.shape[0]
    RB = 1024
    grid = (T // RB,)
    out_shapes = (
        jax.ShapeDtypeStruct((T, 1), jnp.int32),
        jax.ShapeDtypeStruct((T, 1), jnp.int32),
        jax.ShapeDtypeStruct((T, 1), jnp.float32),
        jax.ShapeDtypeStruct((T, 1), jnp.float32),
    )
    o_spec = pl.BlockSpec((RB, 1), lambda i: (i, 0))
    return pl.pallas_call(
        _router_body,
        grid=grid,
        in_specs=[
            pl.BlockSpec((RB, H), lambda i: (i, 0)),
            pl.BlockSpec((E, H), lambda i: (0, 0)),
        ],
        out_specs=(o_spec, o_spec, o_spec, o_spec),
        out_shape=out_shapes,
    )(hidden_states, gate_w)


def _sc_route(eflat, E):
    A = eflat.shape[0]
    NSUB = 16
    C = A // NSUB
    NV = C // 16
    mesh = plsc.VectorSubcoreMesh(core_axis_name="c", subcore_axis_name="s",
                                  num_cores=1)

    @functools.partial(
        pl.kernel, mesh=mesh,
        compiler_params=pltpu.CompilerParams(needs_layout_passes=False),
        out_type=(
            jax.ShapeDtypeStruct((A,), jnp.int32),
            jax.ShapeDtypeStruct((16,), jnp.int32),
            jax.ShapeDtypeStruct((NSUB, 16), jnp.int32),
        ),
        scratch_types=[
            pltpu.VMEM((C,), jnp.int32),
            pltpu.VMEM((C,), jnp.int32),
            pltpu.VMEM((16,), jnp.int32),
            pltpu.VMEM((16,), jnp.int32),
            pltpu.VMEM((NSUB, 16), jnp.int32),
        ],
    )
    def k(e_hbm, pos_hbm, meta_hbm, grid_hbm,
          e_v, pos_v, st_a, st_b, grid_v):
        wid = lax.axis_index("s")
        base = wid * C
        pltpu.sync_copy(e_hbm.at[pl.ds(base, C)], e_v)
        iota = lax.broadcasted_iota(jnp.int32, (16,), 0)
        zero = jnp.zeros((16,), jnp.int32)
        run = [zero for _ in range(E)]
        for v in range(NV):
            ev = e_v[pl.ds(v * 16, 16)]
            rank = zero
            for e in range(E):
                m = ev == e
                pc = jnp.cumsum(jnp.where(m, 1, 0))
                rank = jnp.where(m, run[e] + pc - 1, rank)
                run[e] = run[e] + plsc.all_reduce_population_count(m)
            pos_v[pl.ds(v * 16, 16)] = rank
        cnt16 = zero
        for e in range(E):
            cnt16 = jnp.where(iota == e, run[e], cnt16)
        st_a[...] = cnt16
        for w in range(NSUB):
            @pl.when(wid == w)
            def _(w=w):
                pltpu.sync_copy(st_a, grid_hbm.at[w])
        plsc.subcore_barrier()
        pltpu.sync_copy(grid_hbm, grid_v)
        tot = zero
        prior = zero
        widv = jnp.full((16,), wid, jnp.int32)
        for w in range(NSUB):
            gv = grid_v[w]
            tot = tot + gv
            prior = prior + jnp.where(jnp.full((16,), w, jnp.int32) < widv, gv, zero)
        padded = ((tot + (BT - 1)) // BT) * BT
        csum = jnp.cumsum(padded)
        pstart = csum - padded
        base_v = pstart + prior
        st_a[...] = base_v
        for v in range(NV):
            ev = e_v[pl.ds(v * 16, 16)]
            bse = plsc.load_gather(st_a, [ev])
            pos_v[pl.ds(v * 16, 16)] = pos_v[pl.ds(v * 16, 16)] + bse
        pltpu.sync_copy(pos_v, pos_hbm.at[pl.ds(base, C)])

        @pl.when(wid == 0)
        def _():
            st_b[...] = csum // BT
            pltpu.sync_copy(st_b, meta_hbm)

    pos, meta, _grid = k(eflat)
    return pos, meta


def _sc_gather_scatter(hs, pos, wflat, S):
    T, H = hs.shape
    A = pos.shape[0]
    C = A // _NW
    CH = 64
    nch = C // CH
    mesh = plsc.VectorSubcoreMesh(core_axis_name="c", subcore_axis_name="s")

    @functools.partial(
        pl.kernel, mesh=mesh,
        out_type=(
            jax.ShapeDtypeStruct((S, H), jnp.float32),
            jax.ShapeDtypeStruct((S,), jnp.float32),
        ),
        scratch_types=[
            pltpu.VMEM((CH,), jnp.int32),
            pltpu.VMEM((CH,), jnp.int32),
            pltpu.VMEM((CH,), jnp.float32),
            pltpu.VMEM((CH, H), jnp.float32),
            pltpu.SemaphoreType.DMA,
            pltpu.SemaphoreType.DMA,
        ],
    )
    def k(hs_hbm, pos_hbm, wf_hbm, xs_hbm, ws_hbm,
          idx_v, tok_v, w_v, rows_v, sem_g, sem_s):
        wid = lax.axis_index("s") * 2 + lax.axis_index("c")
        base = wid * C
        iota = lax.broadcasted_iota(jnp.int32, (16,), 0)
        for c in range(nch):
            off = base + c * CH
            pltpu.sync_copy(pos_hbm.at[pl.ds(off, CH)], idx_v)
            pltpu.sync_copy(wf_hbm.at[pl.ds(off, CH)], w_v)
            tb = off - jnp.where(off >= T, T, 0)
            for v in range(CH // 16):
                tok_v[pl.ds(v * 16, 16)] = iota + (tb + v * 16)
            pltpu.async_copy(hs_hbm.at[tok_v], rows_v, sem_g).wait()
            cp_r = pltpu.async_copy(rows_v, xs_hbm.at[idx_v], sem_s)
            cp_w = pltpu.async_copy(w_v, ws_hbm.at[idx_v], sem_g)
            cp_r.wait()
            cp_w.wait()

    return k(hs, pos, wflat)


def _mlp_body(meta_ref, x_ref, w1_ref, w3_ref, w2_ref,
              b13_ref, b2_ref, ws_ref, y_ref):
    I = w1_ref.shape[1]

    @pl.when(pl.program_id(0) < meta_ref[NUM_EXPERTS_C - 1])
    def _():
        x = x_ref[0]
        a = jax.lax.dot_general(
            x, w1_ref[0], (((1,), (1,)), ((), ())),
            preferred_element_type=jnp.float32) + b13_ref[0, :, :I]
        c = jax.lax.dot_general(
            x, w3_ref[0], (((1,), (1,)), ((), ())),
            preferred_element_type=jnp.float32) + b13_ref[0, :, I:]
        h = a * jax.lax.logistic(a) * c
        acc = jax.lax.dot_general(
            h, w2_ref[0], (((1,), (1,)), ((), ())),
            preferred_element_type=jnp.float32)
        y_ref[0] = (acc + b2_ref[0]) * ws_ref[0]


def _run_mlp(x_sorted, w1, w3, w2, w13_bias, w2_bias, w_slot, meta, NBmax):
    E, I, H = w1.shape
    S = NBmax * BT
    x3 = x_sorted.reshape(NBmax, BT, H)
    ws3 = w_slot.reshape(NBmax, BT, 1)

    def live(b, meta_r):
        return jnp.minimum(b, meta_r[E - 1] - 1)

    def expert(b, meta_r):
        bl = live(b, meta_r)
        ex = jnp.int32(0)
        for e in range(E):
            ex = ex + jnp.where(meta_r[e] <= bl, 1, 0).astype(jnp.int32)
        return ex

    def xmap(b, meta_r):
        return (live(b, meta_r), 0, 0)

    def wmap(b, meta_r):
        return (expert(b, meta_r), 0, 0)

    grid_spec = pltpu.PrefetchScalarGridSpec(
        num_scalar_prefetch=1,
        grid=(NBmax,),
        in_specs=[
            pl.BlockSpec((1, BT, H), xmap),
            pl.BlockSpec((1, I, H), wmap),
            pl.BlockSpec((1, I, H), wmap),
            pl.BlockSpec((1, H, I), wmap),
            pl.BlockSpec((1, 1, 2 * I), wmap),
            pl.BlockSpec((1, 1, H), wmap),
            pl.BlockSpec((1, BT, 1), xmap),
        ],
        out_specs=pl.BlockSpec((1, BT, H), xmap),
    )
    y3 = pl.pallas_call(
        _mlp_body,
        grid_spec=grid_spec,
        out_shape=jax.ShapeDtypeStruct((NBmax, BT, H), jnp.float32),
        compiler_params=pltpu.CompilerParams(
            dimension_semantics=("arbitrary",),
            vmem_limit_bytes=120 * 1024 * 1024,
        ),
    )(meta, x3, w1, w3, w2,
      w13_bias.reshape(E, 1, 2 * I), w2_bias.reshape(E, 1, H), ws3)
    return y3.reshape(S, H)


_NW = 32


def _sc_combine(y, pos, T):
    S, H = y.shape
    per_w = T // _NW
    CH = 32
    nch = per_w // CH
    nv = H // 16
    mesh = plsc.VectorSubcoreMesh(core_axis_name="c", subcore_axis_name="s")

    @functools.partial(
        pl.kernel, mesh=mesh,
        out_type=jax.ShapeDtypeStruct((T, H), jnp.float32),
        scratch_types=[
            pltpu.VMEM((CH,), jnp.int32),
            pltpu.VMEM((CH,), jnp.int32),
            pltpu.VMEM((CH, H), jnp.float32),
            pltpu.VMEM((CH, H), jnp.float32),
            pltpu.SemaphoreType.DMA,
            pltpu.SemaphoreType.DMA,
        ],
    )
    def k(y_hbm, p_hbm, out_hbm, i0_v, i1_v, a_v, b_v, sem, sem2):
        wid = lax.axis_index("s") * 2 + lax.axis_index("c")
        base = wid * per_w
        for c in range(nch):
            off = base + c * CH
            pltpu.sync_copy(p_hbm.at[pl.ds(off, CH)], i0_v)
            pltpu.sync_copy(p_hbm.at[pl.ds(T + off, CH)], i1_v)
            cp_a = pltpu.async_copy(y_hbm.at[i0_v], a_v, sem)
            cp_b = pltpu.async_copy(y_hbm.at[i1_v], b_v, sem2)
            cp_a.wait()
            cp_b.wait()

            def row(r, _):
                for j in range(nv):
                    sl = pl.ds(j * 16, 16)
                    a_v[r, sl] = a_v[r, sl] + b_v[r, sl]
                return 0

            lax.fori_loop(0, CH, row, 0)
            pltpu.sync_copy(a_v, out_hbm.at[pl.ds(off, CH)])

    return k(y, pos)


def kernel(hidden_states, gate_w, w1, w3, w2, w13_bias, w2_bias):
    T, H = hidden_states.shape
    E = w1.shape[0]
    A = T * TOP_K_C
    NBmax = A // BT + (E - 1)

    id0, id1, w0, w1r = _run_router(hidden_states, gate_w)
    eflat = jnp.concatenate([id0[:, 0], id1[:, 0]])
    wflat = jnp.concatenate([w0[:, 0], w1r[:, 0]])
    pos, meta = _sc_route(eflat, E)
    x_sorted, w_slot = _sc_gather_scatter(hidden_states, pos, wflat, NBmax * BT)
    y = _run_mlp(x_sorted, w1, w3, w2, w13_bias, w2_bias,
                 w_slot, meta, NBmax)
    return _sc_combine(y, pos, T)


# speedup vs baseline: 1.4734x; 1.0015x over previous
import functools

import jax
import jax.numpy as jnp
from jax import lax
from jax.experimental import pallas as pl
from jax.experimental.pallas import tpu as pltpu
from jax.experimental.pallas import tpu_sc as plsc

NUM_EXPERTS_C = 8
TOP_K_C = 2
BT = 256


def _router_body(x_ref, g_ref, id0_ref, id1_ref, w0_ref, w1_ref):
    x = x_ref[...]
    g = g_ref[...]
    logits = jax.lax.dot_general(
        x, g, (((1,), (1,)), ((), ())), preferred_element_type=jnp.float32)
    rb, e = logits.shape
    iota = jax.lax.broadcasted_iota(jnp.int32, (rb, e), 1)
    m0 = jnp.max(logits, axis=-1, keepdims=True)
    am0 = jnp.min(jnp.where(logits == m0, iota, e), axis=-1, keepdims=True)
    l2 = jnp.where(iota == am0, -jnp.inf, logits)
    m1 = jnp.max(l2, axis=-1, keepdims=True)
    am1 = jnp.min(jnp.where(l2 == m1, iota, e), axis=-1, keepdims=True)
    t = jnp.exp(m1 - m0)
    w0 = 1.0 / (1.0 + t)
    id0_ref[...] = am0
    id1_ref[...] = am1
    w0_ref[...] = w0
    w1_ref[...] = t * w0


def _run_router(hidden_states, gate_w):
    T, H = hidden_states.shape
    E = gate_w.shape[0]
    RB = 1024
    grid = (T // RB,)
    out_shapes = (
        jax.ShapeDtypeStruct((T, 1), jnp.int32),
        jax.ShapeDtypeStruct((T, 1), jnp.int32),
        jax.ShapeDtypeStruct((T, 1), jnp.float32),
        jax.ShapeDtypeStruct((T, 1), jnp.float32),
    )
    o_spec = pl.BlockSpec((RB, 1), lambda i: (i, 0))
    return pl.pallas_call(
        _router_body,
        grid=grid,
        in_specs=[
            pl.BlockSpec((RB, H), lambda i: (i, 0)),
            pl.BlockSpec((E, H), lambda i: (0, 0)),
        ],
        out_specs=(o_spec, o_spec, o_spec, o_spec),
        out_shape=out_shapes,
    )(hidden_states, gate_w)


def _sc_route(eflat, E):
    A = eflat.shape[0]
    NSUB = 16
    C = A // NSUB
    NV = C // 16
    mesh = plsc.VectorSubcoreMesh(core_axis_name="c", subcore_axis_name="s",
                                  num_cores=1)

    @functools.partial(
        pl.kernel, mesh=mesh,
        compiler_params=pltpu.CompilerParams(needs_layout_passes=False),
        out_type=(
            jax.ShapeDtypeStruct((A,), jnp.int32),
            jax.ShapeDtypeStruct((16,), jnp.int32),
            jax.ShapeDtypeStruct((NSUB, 16), jnp.int32),
        ),
        scratch_types=[
            pltpu.VMEM((C,), jnp.int32),
            pltpu.VMEM((C,), jnp.int32),
            pltpu.VMEM((16,), jnp.int32),
            pltpu.VMEM((16,), jnp.int32),
            pltpu.VMEM((NSUB, 16), jnp.int32),
        ],
    )
    def k(e_hbm, pos_hbm, meta_hbm, grid_hbm,
          e_v, pos_v, st_a, st_b, grid_v):
        wid = lax.axis_index("s")
        base = wid * C
        pltpu.sync_copy(e_hbm.at[pl.ds(base, C)], e_v)
        iota = lax.broadcasted_iota(jnp.int32, (16,), 0)
        zero = jnp.zeros((16,), jnp.int32)
        run = [zero for _ in range(E)]
        for v in range(NV):
            ev = e_v[pl.ds(v * 16, 16)]
            rank = zero
            for e in range(E):
                m = ev == e
                pc = jnp.cumsum(jnp.where(m, 1, 0))
                rank = jnp.where(m, run[e] + pc - 1, rank)
                run[e] = run[e] + plsc.all_reduce_population_count(m)
            pos_v[pl.ds(v * 16, 16)] = rank
        cnt16 = zero
        for e in range(E):
            cnt16 = jnp.where(iota == e, run[e], cnt16)
        st_a[...] = cnt16
        for w in range(NSUB):
            @pl.when(wid == w)
            def _(w=w):
                pltpu.sync_copy(st_a, grid_hbm.at[w])
        plsc.subcore_barrier()
        pltpu.sync_copy(grid_hbm, grid_v)
        tot = zero
        prior = zero
        widv = jnp.full((16,), wid, jnp.int32)
        for w in range(NSUB):
            gv = grid_v[w]
            tot = tot + gv
            prior = prior + jnp.where(jnp.full((16,), w, jnp.int32) < widv, gv, zero)
        padded = ((tot + (BT - 1)) // BT) * BT
        csum = jnp.cumsum(padded)
        pstart = csum - padded
        base_v = pstart + prior
        st_a[...] = base_v
        for v in range(NV):
            ev = e_v[pl.ds(v * 16, 16)]
            bse = plsc.load_gather(st_a, [ev])
            pos_v[pl.ds(v * 16, 16)] = pos_v[pl.ds(v * 16, 16)] + bse
        pltpu.sync_copy(pos_v, pos_hbm.at[pl.ds(base, C)])

        @pl.when(wid == 0)
        def _():
            st_b[...] = csum // BT
            pltpu.sync_copy(st_b, meta_hbm)

    pos, meta, _grid = k(eflat)
    return pos, meta


def _sc_gather_scatter(hs, pos, wflat, S):
    T, H = hs.shape
    A = pos.shape[0]
    C = A // _NW
    CH = 32
    nch = C // CH
    mesh = plsc.VectorSubcoreMesh(core_axis_name="c", subcore_axis_name="s")

    @functools.partial(
        pl.kernel, mesh=mesh,
        out_type=(
            jax.ShapeDtypeStruct((S, H), jnp.float32),
            jax.ShapeDtypeStruct((S,), jnp.float32),
        ),
        scratch_types=[
            pltpu.VMEM((2, CH), jnp.int32),
            pltpu.VMEM((2, CH), jnp.int32),
            pltpu.VMEM((2, CH), jnp.float32),
            pltpu.VMEM((CH, H), jnp.float32),
            pltpu.VMEM((CH, H), jnp.float32),
            pltpu.SemaphoreType.DMA,
            pltpu.SemaphoreType.DMA,
            pltpu.SemaphoreType.DMA,
        ],
    )
    def k(hs_hbm, pos_hbm, wf_hbm, xs_hbm, ws_hbm,
          idx_v, tok_v, w_v, rows_v0, rows_v1, sem_g, sem_s, sem_w):
        wid = lax.axis_index("s") * 2 + lax.axis_index("c")
        base = wid * C
        iota = lax.broadcasted_iota(jnp.int32, (16,), 0)
        rows = (rows_v0, rows_v1)

        def load_and_gather(c):
            b = c % 2
            off = base + c * CH
            pltpu.sync_copy(pos_hbm.at[pl.ds(off, CH)], idx_v.at[b])
            pltpu.sync_copy(wf_hbm.at[pl.ds(off, CH)], w_v.at[b])
            tb = off - jnp.where(off >= T, T, 0)
            for v in range(CH // 16):
                tok_v[b, pl.ds(v * 16, 16)] = iota + (tb + v * 16)
            return pltpu.async_copy(hs_hbm.at[tok_v.at[b]], rows[b], sem_g)

        g = load_and_gather(0)
        prev = None
        for c in range(nch):
            b = c % 2
            g.wait()
            if prev is not None:
                prev[0].wait()
                prev[1].wait()
            if c + 1 < nch:
                g = load_and_gather(c + 1)
            cp_r = pltpu.async_copy(rows[b], xs_hbm.at[idx_v.at[b]], sem_s)
            cp_w = pltpu.async_copy(w_v.at[b], ws_hbm.at[idx_v.at[b]], sem_w)
            prev = (cp_r, cp_w)
        prev[0].wait()
        prev[1].wait()

    return k(hs, pos, wflat)


def _mlp_body(meta_ref, x_ref, w1_ref, w3_ref, w2_ref,
              b13_ref, b2_ref, ws_ref, y_ref):
    I = w1_ref.shape[1]

    @pl.when(pl.program_id(0) < meta_ref[NUM_EXPERTS_C - 1])
    def _():
        x = x_ref[0]
        a = jax.lax.dot_general(
            x, w1_ref[0], (((1,), (1,)), ((), ())),
            preferred_element_type=jnp.float32) + b13_ref[0, :, :I]
        c = jax.lax.dot_general(
            x, w3_ref[0], (((1,), (1,)), ((), ())),
            preferred_element_type=jnp.float32) + b13_ref[0, :, I:]
        h = a * jax.lax.logistic(a) * c
        acc = jax.lax.dot_general(
            h, w2_ref[0], (((1,), (1,)), ((), ())),
            preferred_element_type=jnp.float32)
        y_ref[0] = (acc + b2_ref[0]) * ws_ref[0]


def _run_mlp(x_sorted, w1, w3, w2, w13_bias, w2_bias, w_slot, meta, NBmax):
    E, I, H = w1.shape
    S = NBmax * BT
    x3 = x_sorted.reshape(NBmax, BT, H)
    ws3 = w_slot.reshape(NBmax, BT, 1)

    def live(b, meta_r):
        return jnp.minimum(b, meta_r[E - 1] - 1)

    def expert(b, meta_r):
        bl = live(b, meta_r)
        ex = jnp.int32(0)
        for e in range(E):
            ex = ex + jnp.where(meta_r[e] <= bl, 1, 0).astype(jnp.int32)
        return ex

    def xmap(b, meta_r):
        return (live(b, meta_r), 0, 0)

    def wmap(b, meta_r):
        return (expert(b, meta_r), 0, 0)

    grid_spec = pltpu.PrefetchScalarGridSpec(
        num_scalar_prefetch=1,
        grid=(NBmax,),
        in_specs=[
            pl.BlockSpec((1, BT, H), xmap),
            pl.BlockSpec((1, I, H), wmap),
            pl.BlockSpec((1, I, H), wmap),
            pl.BlockSpec((1, H, I), wmap),
            pl.BlockSpec((1, 1, 2 * I), wmap),
            pl.BlockSpec((1, 1, H), wmap),
            pl.BlockSpec((1, BT, 1), xmap),
        ],
        out_specs=pl.BlockSpec((1, BT, H), xmap),
    )
    y3 = pl.pallas_call(
        _mlp_body,
        grid_spec=grid_spec,
        out_shape=jax.ShapeDtypeStruct((NBmax, BT, H), jnp.float32),
        compiler_params=pltpu.CompilerParams(
            dimension_semantics=("arbitrary",),
            vmem_limit_bytes=120 * 1024 * 1024,
        ),
    )(meta, x3, w1, w3, w2,
      w13_bias.reshape(E, 1, 2 * I), w2_bias.reshape(E, 1, H), ws3)
    return y3.reshape(S, H)


_NW = 32


def _sc_combine(y, pos, T):
    S, H = y.shape
    per_w = T // _NW
    CH = 32
    nch = per_w // CH
    nv = H // 16
    mesh = plsc.VectorSubcoreMesh(core_axis_name="c", subcore_axis_name="s")

    @functools.partial(
        pl.kernel, mesh=mesh,
        out_type=jax.ShapeDtypeStruct((T, H), jnp.float32),
        scratch_types=[
            pltpu.VMEM((CH,), jnp.int32),
            pltpu.VMEM((CH,), jnp.int32),
            pltpu.VMEM((CH, H), jnp.float32),
            pltpu.VMEM((CH, H), jnp.float32),
            pltpu.SemaphoreType.DMA,
            pltpu.SemaphoreType.DMA,
        ],
    )
    def k(y_hbm, p_hbm, out_hbm, i0_v, i1_v, a_v, b_v, sem, sem2):
        wid = lax.axis_index("s") * 2 + lax.axis_index("c")
        base = wid * per_w
        for c in range(nch):
            off = base + c * CH
            pltpu.sync_copy(p_hbm.at[pl.ds(off, CH)], i0_v)
            pltpu.sync_copy(p_hbm.at[pl.ds(T + off, CH)], i1_v)
            cp_a = pltpu.async_copy(y_hbm.at[i0_v], a_v, sem)
            cp_b = pltpu.async_copy(y_hbm.at[i1_v], b_v, sem2)
            cp_a.wait()
            cp_b.wait()

            def row(r, _):
                for j in range(nv):
                    sl = pl.ds(j * 16, 16)
                    a_v[r, sl] = a_v[r, sl] + b_v[r, sl]
                return 0

            lax.fori_loop(0, CH, row, 0)
            pltpu.sync_copy(a_v, out_hbm.at[pl.ds(off, CH)])

    return k(y, pos)


def kernel(hidden_states, gate_w, w1, w3, w2, w13_bias, w2_bias):
    T, H = hidden_states.shape
    E = w1.shape[0]
    A = T * TOP_K_C
    NBmax = A // BT + (E - 1)

    id0, id1, w0, w1r = _run_router(hidden_states, gate_w)
    eflat = jnp.concatenate([id0[:, 0], id1[:, 0]])
    wflat = jnp.concatenate([w0[:, 0], w1r[:, 0]])
    pos, meta = _sc_route(eflat, E)
    x_sorted, w_slot = _sc_gather_scatter(hidden_states, pos, wflat, NBmax * BT)
    y = _run_mlp(x_sorted, w1, w3, w2, w13_bias, w2_bias,
                 w_slot, meta, NBmax)
    return _sc_combine(y, pos, T)
